# baseline (device time: 116295 ns/iter reference)
import jax
import jax.numpy as jnp
from jax import lax
from jax.experimental import pallas as pl
from jax.experimental.pallas import tpu as pltpu

N_DEV = 4
W_CHUNKS = 8


def kernel(x, w_mat, scale_x, scale_w):
    m_per, k = x.shape
    n = w_mat.shape[1]
    n_per = n // N_DEV
    h_per = m_per // 2
    k_chunk = k // W_CHUNKS

    s = (scale_x * scale_w).astype(jnp.float32)

    def body(x_ref, w_hbm, s_ref, out_ref, cw_ref, ccw_ref, w8_ref,
             wstage_ref, acc_ref, cw_send, cw_recv, ccw_send, ccw_recv,
             copy_sems, wdma_sems):
        my_pos = lax.axis_index("i")
        left = (my_pos - 1) % N_DEV
        right = (my_pos + 1) % N_DEV

        barrier_sem = pltpu.get_barrier_semaphore()
        for nbr in [left, right]:
            pl.semaphore_signal(
                barrier_sem, inc=1,
                device_id=(nbr,), device_id_type=pl.DeviceIdType.MESH,
            )
        pl.semaphore_wait(barrier_sem, 2)

        scale = s_ref[0]

        cw_ref[0] = x_ref[:h_per, :].astype(jnp.float8_e5m2)
        ccw_ref[0] = x_ref[h_per:, :].astype(jnp.float8_e5m2)

        pending = []

        def compute_store(src, row0, sem_slot):
            acc_ref[sem_slot] = (
                jnp.dot(src, w8_ref[...], preferred_element_type=jnp.float32)
                * scale
            )
            cp = pltpu.make_async_copy(
                acc_ref.at[sem_slot],
                out_ref.at[pl.ds(row0, h_per), :],
                copy_sems.at[sem_slot],
            )
            cp.start()
            pending.append(cp)

        for hop in range(N_DEV - 1):
            s_slot = hop % 2
            r_slot = (hop + 1) % 2
            cw_rdma = pltpu.make_async_remote_copy(
                src_ref=cw_ref.at[s_slot],
                dst_ref=cw_ref.at[r_slot],
                send_sem=cw_send.at[s_slot],
                recv_sem=cw_recv.at[r_slot],
                device_id=(right,),
                device_id_type=pl.DeviceIdType.MESH,
            )
            ccw_rdma = pltpu.make_async_remote_copy(
                src_ref=ccw_ref.at[s_slot],
                dst_ref=ccw_ref.at[r_slot],
                send_sem=ccw_send.at[s_slot],
                recv_sem=ccw_recv.at[r_slot],
                device_id=(left,),
                device_id_type=pl.DeviceIdType.MESH,
            )
            cw_rdma.start()
            ccw_rdma.start()

            if hop == 0:
                dmas = []
                for c in range(W_CHUNKS):
                    d = pltpu.make_async_copy(
                        w_hbm.at[pl.ds(c * k_chunk, k_chunk),
                                 pl.ds(my_pos * n_per, n_per)],
                        wstage_ref.at[c % 2],
                        wdma_sems.at[c % 2],
                    )
                    d.start()
                    dmas.append(d)
                    if c >= 1:
                        dmas[c - 1].wait()
                        w8_ref[pl.ds((c - 1) * k_chunk, k_chunk), :] = (
                            wstage_ref[(c - 1) % 2].astype(jnp.float8_e5m2)
                        )
                dmas[W_CHUNKS - 1].wait()
                w8_ref[pl.ds((W_CHUNKS - 1) * k_chunk, k_chunk), :] = (
                    wstage_ref[(W_CHUNKS - 1) % 2].astype(jnp.float8_e5m2)
                )
                compute_store(cw_ref[0], my_pos * m_per, 0)
                compute_store(ccw_ref[0], my_pos * m_per + h_per, 1)
            else:
                while pending:
                    pending.pop(0).wait()
                o_cw = (my_pos - hop) % N_DEV
                o_ccw = (my_pos + hop) % N_DEV
                compute_store(cw_ref[s_slot], o_cw * m_per, 0)
                compute_store(ccw_ref[s_slot], o_ccw * m_per + h_per, 1)

            cw_rdma.wait()
            ccw_rdma.wait()

        while pending:
            pending.pop(0).wait()
        o_cw = (my_pos - (N_DEV - 1)) % N_DEV
        o_ccw = (my_pos + (N_DEV - 1)) % N_DEV
        compute_store(cw_ref[1], o_cw * m_per, 0)
        compute_store(ccw_ref[1], o_ccw * m_per + h_per, 1)
        while pending:
            pending.pop(0).wait()

    return pl.pallas_call(
        body,
        out_shape=jax.ShapeDtypeStruct((N_DEV * m_per, n_per), jnp.float32),
        in_specs=[
            pl.BlockSpec(memory_space=pltpu.VMEM),
            pl.BlockSpec(memory_space=pl.ANY),
            pl.BlockSpec(memory_space=pltpu.SMEM),
        ],
        out_specs=pl.BlockSpec(memory_space=pl.ANY),
        scratch_shapes=[
            pltpu.VMEM((2, h_per, k), jnp.float8_e5m2),
            pltpu.VMEM((2, h_per, k), jnp.float8_e5m2),
            pltpu.VMEM((k, n_per), jnp.float8_e5m2),
            pltpu.VMEM((2, k_chunk, n_per), jnp.float32),
            pltpu.VMEM((2, h_per, n_per), jnp.float32),
            pltpu.SemaphoreType.DMA((2,)),
            pltpu.SemaphoreType.DMA((2,)),
            pltpu.SemaphoreType.DMA((2,)),
            pltpu.SemaphoreType.DMA((2,)),
            pltpu.SemaphoreType.DMA((2,)),
            pltpu.SemaphoreType.DMA((2,)),
        ],
        compiler_params=pltpu.CompilerParams(
            collective_id=0, vmem_limit_bytes=60 * 1024 * 1024
        ),
    )(x, w_mat, s)
